# baseline (speedup 1.0000x reference)
import functools

import jax
import jax.numpy as jnp
from jax import lax
from jax.experimental import pallas as pl
from jax.experimental.pallas import tpu as pltpu
from jax.experimental.pallas import tpu_sc as plsc

B = 16384
D = 32
F = 4 * D
V = 16

_info = plsc.get_sparse_core_info()
_NC, _NS = _info.num_cores, _info.num_subcores
NW = _NC * _NS
BPW = B // NW
NG = BPW // V


def _sc_gather(users, items, mf_u3, mf_i3, mlp_u3, mlp_i3):
    mesh = plsc.VectorSubcoreMesh(core_axis_name="c", subcore_axis_name="s")

    @functools.partial(
        pl.kernel, mesh=mesh,
        out_type=jax.ShapeDtypeStruct((B, F), jnp.float32),
        scratch_types=[
            pltpu.VMEM((BPW,), jnp.int32),
            pltpu.VMEM((BPW,), jnp.int32),
            pltpu.VMEM((V, 8, D), jnp.float32),
            pltpu.VMEM((V, 8, D), jnp.float32),
            pltpu.VMEM((V, 8, D), jnp.float32),
            pltpu.VMEM((V, 8, D), jnp.float32),
            pltpu.VMEM((V, F), jnp.float32),
            pltpu.SemaphoreType.DMA,
        ],
        compiler_params=pltpu.CompilerParams(needs_layout_passes=False),
    )
    def k(users_h, items_h, mfu_h, mfi_h, mlpu_h, mlpi_h, out_h,
          u_v, i_v, r_mfu, r_mfi, r_mlpu, r_mlpi, stage, sem):
        wid = lax.axis_index("s") * _NC + lax.axis_index("c")
        base = wid * BPW
        pltpu.sync_copy(users_h.at[pl.ds(base, BPW)], u_v)
        pltpu.sync_copy(items_h.at[pl.ds(base, BPW)], i_v)

        tabs = [(mfu_h, r_mfu, 0), (mfi_h, r_mfi, 1),
                (mlpu_h, r_mlpu, 2), (mlpi_h, r_mlpi, 3)]
        iot = lax.iota(jnp.int32, V)

        def grp(g, _):
            ug = u_v[pl.ds(g * V, V)]
            ig = i_v[pl.ds(g * V, V)]
            gu = (ug >> 3) << 3
            gi = (ig >> 3) << 3
            su = ug & 7
            si = ig & 7
            cps = []
            for l in range(V):
                for tab, r, t in tabs:
                    row0 = gu[l] if t in (0, 2) else gi[l]
                    row0 = pl.multiple_of(row0, 8)
                    cps.append(pltpu.async_copy(
                        tab.at[pl.ds(row0, 8)], r.at[l], sem))
            for cp in cps:
                cp.wait()
            for l in range(V):
                lvec = jnp.full((V,), l, dtype=jnp.int32)
                for tab, r, t in tabs:
                    s = su[l] if t in (0, 2) else si[l]
                    svec = jnp.full((V,), s, dtype=jnp.int32)
                    for h in range(D // V):
                        vals = plsc.load_gather(
                            r, [lvec, svec, h * V + iot])
                        stage[l, pl.ds(t * D + h * V, V)] = vals
            pltpu.sync_copy(stage, out_h.at[pl.ds(base + g * V, V)])
            return 0

        lax.fori_loop(0, NG, grp, 0)

    return k(users, items, mf_u3, mf_i3, mlp_u3, mlp_i3)


def _tc_body(emb, w1a, w1b, b1r, w2, b2r, w3, b3r, woa, wob, bor, out):
    f32 = jnp.float32
    e = emb[:]
    gmf = e[:, :D] * e[:, D:2 * D]
    h = jnp.dot(e[:, 2 * D:3 * D], w1a[:], preferred_element_type=f32)
    h = h + jnp.dot(e[:, 3 * D:], w1b[:], preferred_element_type=f32)
    h = jnp.maximum(h + b1r[:], 0.0)
    h = jnp.maximum(jnp.dot(h, w2[:], preferred_element_type=f32) + b2r[:], 0.0)
    h = jnp.maximum(jnp.dot(h, w3[:], preferred_element_type=f32) + b3r[:], 0.0)
    logit = (jnp.dot(gmf, woa[:], preferred_element_type=f32)
             + jnp.dot(h, wob[:], preferred_element_type=f32) + bor[:])
    out[:] = jax.nn.sigmoid(logit)


def _tc_mlp(emb, W1, b1, W2, b2, W3, b3, Wo, bo):
    bs = 2048
    grid = (B // bs,)
    w1a, w1b = W1[:D], W1[D:]
    woa, wob = Wo[:D], Wo[D:]
    b1r = b1.reshape(1, -1)
    b2r = b2.reshape(1, -1)
    b3r = b3.reshape(1, -1)
    bor = bo.reshape(1, 1)

    def full(a):
        return pl.BlockSpec(a.shape, lambda i: (0,) * a.ndim)

    return pl.pallas_call(
        _tc_body,
        grid=grid,
        in_specs=[
            pl.BlockSpec((bs, F), lambda i: (i, 0)),
            full(w1a), full(w1b), full(b1r),
            full(W2), full(b2r),
            full(W3), full(b3r),
            full(woa), full(wob), full(bor),
        ],
        out_specs=pl.BlockSpec((bs, 1), lambda i: (i, 0)),
        out_shape=jax.ShapeDtypeStruct((B, 1), jnp.float32),
    )(emb, w1a, w1b, b1r, W2, b2r, W3, b3r, woa, wob, bor)


def kernel(users, items, mf_u, mf_i, mlp_u, mlp_i, W1, b1, W2, b2, W3, b3,
           Wo, bo):
    emb = _sc_gather(users, items, mf_u, mf_i, mlp_u, mlp_i)
    return _tc_mlp(emb, W1, b1, W2, b2, W3, b3, Wo, bo)


# speedup vs baseline: 2.5065x; 2.5065x over previous
import functools

import jax
import jax.numpy as jnp
from jax import lax
from jax.experimental import pallas as pl
from jax.experimental.pallas import tpu as pltpu
from jax.experimental.pallas import tpu_sc as plsc

B = 16384
D = 32
F = 4 * D
V = 16
W = 128

_info = plsc.get_sparse_core_info()
_NC, _NS = _info.num_cores, _info.num_subcores
NW = _NC * _NS
BPW = B // NW
NG = BPW // V


def _sc_gather(users, items, mf_uT, mf_iT, mlp_uT, mlp_iT):
    mesh = plsc.VectorSubcoreMesh(core_axis_name="c", subcore_axis_name="s")

    @functools.partial(
        pl.kernel, mesh=mesh,
        out_type=jax.ShapeDtypeStruct((B, F), jnp.float32),
        scratch_types=[
            pltpu.VMEM((BPW,), jnp.int32),
            pltpu.VMEM((BPW,), jnp.int32),
            pltpu.VMEM((4, D, W), jnp.float32),
            pltpu.VMEM((4, D, W), jnp.float32),
            pltpu.VMEM((4, D, W), jnp.float32),
            pltpu.VMEM((4, D, W), jnp.float32),
            pltpu.VMEM((V, F), jnp.float32),
            pltpu.SemaphoreType.DMA,
        ],
        compiler_params=pltpu.CompilerParams(needs_layout_passes=False),
    )
    def k(users_h, items_h, mfu_h, mfi_h, mlpu_h, mlpi_h, out_h,
          u_v, i_v, r_mfu, r_mfi, r_mlpu, r_mlpi, stage, sem):
        wid = lax.axis_index("s") * _NC + lax.axis_index("c")
        base = wid * BPW
        pltpu.sync_copy(users_h.at[pl.ds(base, BPW)], u_v)
        pltpu.sync_copy(items_h.at[pl.ds(base, BPW)], i_v)

        tabs = [(mfu_h, r_mfu, 0), (mfi_h, r_mfi, 1),
                (mlpu_h, r_mlpu, 2), (mlpi_h, r_mlpi, 3)]
        iot = lax.iota(jnp.int32, V)

        def grp(g, _):
            ug = u_v[pl.ds(g * V, V)]
            ig = i_v[pl.ds(g * V, V)]
            cu = (ug // W) * W
            ci = (ig // W) * W
            lu = ug - cu
            li = ig - ci
            for sub in range(4):
                cps = []
                for j in range(4):
                    l = sub * 4 + j
                    for tab, r, t in tabs:
                        col = cu[l] if t in (0, 2) else ci[l]
                        col = pl.multiple_of(col, W)
                        cps.append(pltpu.async_copy(
                            tab.at[:, pl.ds(col, W)], r.at[j], sem))
                for cp in cps:
                    cp.wait()
                for j in range(4):
                    l = sub * 4 + j
                    jvec = jnp.full((V,), j, dtype=jnp.int32)
                    for tab, r, t in tabs:
                        lane = lu[l] if t in (0, 2) else li[l]
                        svec = jnp.full((V,), lane, dtype=jnp.int32)
                        for h in range(D // V):
                            vals = plsc.load_gather(
                                r, [jvec, h * V + iot, svec])
                            stage[l, pl.ds(t * D + h * V, V)] = vals
            pltpu.sync_copy(stage, out_h.at[pl.ds(base + g * V, V)])
            return 0

        lax.fori_loop(0, NG, grp, 0)

    return k(users, items, mf_uT, mf_iT, mlp_uT, mlp_iT)


def _tc_body(emb, w1a, w1b, b1r, w2, b2r, w3, b3r, woa, wob, bor, out):
    f32 = jnp.float32
    e = emb[:]
    gmf = e[:, :D] * e[:, D:2 * D]
    h = jnp.dot(e[:, 2 * D:3 * D], w1a[:], preferred_element_type=f32)
    h = h + jnp.dot(e[:, 3 * D:], w1b[:], preferred_element_type=f32)
    h = jnp.maximum(h + b1r[:], 0.0)
    h = jnp.maximum(jnp.dot(h, w2[:], preferred_element_type=f32) + b2r[:], 0.0)
    h = jnp.maximum(jnp.dot(h, w3[:], preferred_element_type=f32) + b3r[:], 0.0)
    logit = (jnp.dot(gmf, woa[:], preferred_element_type=f32)
             + jnp.dot(h, wob[:], preferred_element_type=f32) + bor[:])
    out[:] = jax.nn.sigmoid(logit)


def _tc_mlp(emb, W1, b1, W2, b2, W3, b3, Wo, bo):
    bs = 2048
    grid = (B // bs,)
    w1a, w1b = W1[:D], W1[D:]
    woa, wob = Wo[:D], Wo[D:]
    b1r = b1.reshape(1, -1)
    b2r = b2.reshape(1, -1)
    b3r = b3.reshape(1, -1)
    bor = bo.reshape(1, 1)

    def full(a):
        return pl.BlockSpec(a.shape, lambda i: (0,) * a.ndim)

    return pl.pallas_call(
        _tc_body,
        grid=grid,
        in_specs=[
            pl.BlockSpec((bs, F), lambda i: (i, 0)),
            full(w1a), full(w1b), full(b1r),
            full(W2), full(b2r),
            full(W3), full(b3r),
            full(woa), full(wob), full(bor),
        ],
        out_specs=pl.BlockSpec((bs, 1), lambda i: (i, 0)),
        out_shape=jax.ShapeDtypeStruct((B, 1), jnp.float32),
    )(emb, w1a, w1b, b1r, W2, b2r, W3, b3r, woa, wob, bor)


def kernel(users, items, mf_u, mf_i, mlp_u, mlp_i, W1, b1, W2, b2, W3, b3,
           Wo, bo):
    emb = _sc_gather(users, items, mf_u.T, mf_i.T, mlp_u.T, mlp_i.T)
    return _tc_mlp(emb, W1, b1, W2, b2, W3, b3, Wo, bo)


# speedup vs baseline: 2.5465x; 1.0160x over previous
import functools

import jax
import jax.numpy as jnp
from jax import lax
from jax.experimental import pallas as pl
from jax.experimental.pallas import tpu as pltpu
from jax.experimental.pallas import tpu_sc as plsc

B = 16384
D = 32
F = 4 * D
V = 16
W = 128

_info = plsc.get_sparse_core_info()
_NC, _NS = _info.num_cores, _info.num_subcores
NW = _NC * _NS
BPW = B // NW
NG = BPW // V


def _sc_gather(users, items, mf_uT, mf_iT, mlp_uT, mlp_iT):
    mesh = plsc.VectorSubcoreMesh(core_axis_name="c", subcore_axis_name="s")

    @functools.partial(
        pl.kernel, mesh=mesh,
        out_type=jax.ShapeDtypeStruct((B, F), jnp.float32),
        scratch_types=[
            pltpu.VMEM((BPW,), jnp.int32),
            pltpu.VMEM((BPW,), jnp.int32),
            pltpu.VMEM((4, D, W), jnp.float32),
            pltpu.VMEM((4, D, W), jnp.float32),
            pltpu.VMEM((4, D, W), jnp.float32),
            pltpu.VMEM((4, D, W), jnp.float32),
            pltpu.VMEM((V, F), jnp.float32),
            pltpu.SemaphoreType.DMA,
            pltpu.SemaphoreType.DMA,
        ],
        compiler_params=pltpu.CompilerParams(needs_layout_passes=False),
    )
    def k(users_h, items_h, mfu_h, mfi_h, mlpu_h, mlpi_h, out_h,
          u_v, i_v, r_mfu, r_mfi, r_mlpu, r_mlpi, stage, sem0, sem1):
        wid = lax.axis_index("s") * _NC + lax.axis_index("c")
        base = wid * BPW
        pltpu.sync_copy(users_h.at[pl.ds(base, BPW)], u_v)
        pltpu.sync_copy(items_h.at[pl.ds(base, BPW)], i_v)

        tabs = [(mfu_h, r_mfu, 0), (mfi_h, r_mfi, 1),
                (mlpu_h, r_mlpu, 2), (mlpi_h, r_mlpi, 3)]
        iot = lax.iota(jnp.int32, V)

        sems = (sem0, sem1)

        def grp(g, _):
            ug = u_v[pl.ds(g * V, V)]
            ig = i_v[pl.ds(g * V, V)]
            cu = (ug // W) * W
            ci = (ig // W) * W
            lu = ug - cu
            li = ig - ci

            def fire(u):
                slot = u % 2
                cps = []
                for j in range(2):
                    l = u * 2 + j
                    for tab, r, t in tabs:
                        col = cu[l] if t in (0, 2) else ci[l]
                        col = pl.multiple_of(col, W)
                        cps.append(pltpu.async_copy(
                            tab.at[:, pl.ds(col, W)], r.at[slot * 2 + j],
                            sems[slot]))
                return cps

            def extract(u):
                slot = u % 2
                for j in range(2):
                    l = u * 2 + j
                    jvec = jnp.full((V,), slot * 2 + j, dtype=jnp.int32)
                    for tab, r, t in tabs:
                        lane = lu[l] if t in (0, 2) else li[l]
                        svec = jnp.full((V,), lane, dtype=jnp.int32)
                        for h in range(D // V):
                            vals = plsc.load_gather(
                                r, [jvec, h * V + iot, svec])
                            stage[l, pl.ds(t * D + h * V, V)] = vals

            pend = fire(0)
            for u in range(V // 2):
                nxt = fire(u + 1) if u < V // 2 - 1 else None
                for cp in pend:
                    cp.wait()
                extract(u)
                pend = nxt
            pltpu.sync_copy(stage, out_h.at[pl.ds(base + g * V, V)])
            return 0

        lax.fori_loop(0, NG, grp, 0)

    return k(users, items, mf_uT, mf_iT, mlp_uT, mlp_iT)


def _tc_body(emb, w1a, w1b, b1r, w2, b2r, w3, b3r, woa, wob, bor, out):
    f32 = jnp.float32
    e = emb[:]
    gmf = e[:, :D] * e[:, D:2 * D]
    h = jnp.dot(e[:, 2 * D:3 * D], w1a[:], preferred_element_type=f32)
    h = h + jnp.dot(e[:, 3 * D:], w1b[:], preferred_element_type=f32)
    h = jnp.maximum(h + b1r[:], 0.0)
    h = jnp.maximum(jnp.dot(h, w2[:], preferred_element_type=f32) + b2r[:], 0.0)
    h = jnp.maximum(jnp.dot(h, w3[:], preferred_element_type=f32) + b3r[:], 0.0)
    logit = (jnp.dot(gmf, woa[:], preferred_element_type=f32)
             + jnp.dot(h, wob[:], preferred_element_type=f32) + bor[:])
    out[:] = jax.nn.sigmoid(logit)


def _tc_mlp(emb, W1, b1, W2, b2, W3, b3, Wo, bo):
    bs = 2048
    grid = (B // bs,)
    w1a, w1b = W1[:D], W1[D:]
    woa, wob = Wo[:D], Wo[D:]
    b1r = b1.reshape(1, -1)
    b2r = b2.reshape(1, -1)
    b3r = b3.reshape(1, -1)
    bor = bo.reshape(1, 1)

    def full(a):
        return pl.BlockSpec(a.shape, lambda i: (0,) * a.ndim)

    return pl.pallas_call(
        _tc_body,
        grid=grid,
        in_specs=[
            pl.BlockSpec((bs, F), lambda i: (i, 0)),
            full(w1a), full(w1b), full(b1r),
            full(W2), full(b2r),
            full(W3), full(b3r),
            full(woa), full(wob), full(bor),
        ],
        out_specs=pl.BlockSpec((bs, 1), lambda i: (i, 0)),
        out_shape=jax.ShapeDtypeStruct((B, 1), jnp.float32),
    )(emb, w1a, w1b, b1r, W2, b2r, W3, b3r, woa, wob, bor)


def kernel(users, items, mf_u, mf_i, mlp_u, mlp_i, W1, b1, W2, b2, W3, b3,
           Wo, bo):
    emb = _sc_gather(users, items, mf_u.T, mf_i.T, mlp_u.T, mlp_i.T)
    return _tc_mlp(emb, W1, b1, W2, b2, W3, b3, Wo, bo)


# speedup vs baseline: 2.6259x; 1.0312x over previous
import functools

import jax
import jax.numpy as jnp
from jax import lax
from jax.experimental import pallas as pl
from jax.experimental.pallas import tpu as pltpu
from jax.experimental.pallas import tpu_sc as plsc

B = 16384
D = 32
F = 4 * D
V = 16
W = 128

_info = plsc.get_sparse_core_info()
_NC, _NS = _info.num_cores, _info.num_subcores
NW = _NC * _NS
BPW = B // NW
NG = BPW // V


def _sc_gather(users, items, mf_uT, mf_iT, mlp_uT, mlp_iT):
    mesh = plsc.VectorSubcoreMesh(core_axis_name="c", subcore_axis_name="s")

    @functools.partial(
        pl.kernel, mesh=mesh,
        out_type=jax.ShapeDtypeStruct((B, F), jnp.float32),
        scratch_types=[
            pltpu.VMEM((BPW,), jnp.int32),
            pltpu.VMEM((BPW,), jnp.int32),
            pltpu.VMEM((6, D, W), jnp.float32),
            pltpu.VMEM((6, D, W), jnp.float32),
            pltpu.VMEM((6, D, W), jnp.float32),
            pltpu.VMEM((6, D, W), jnp.float32),
            pltpu.VMEM((V, F), jnp.float32),
            pltpu.SemaphoreType.DMA,
            pltpu.SemaphoreType.DMA,
            pltpu.SemaphoreType.DMA,
        ],
        compiler_params=pltpu.CompilerParams(needs_layout_passes=False),
    )
    def k(users_h, items_h, mfu_h, mfi_h, mlpu_h, mlpi_h, out_h,
          u_v, i_v, r_mfu, r_mfi, r_mlpu, r_mlpi, stage, sem0, sem1, sem2):
        wid = lax.axis_index("s") * _NC + lax.axis_index("c")
        base = wid * BPW
        pltpu.sync_copy(users_h.at[pl.ds(base, BPW)], u_v)
        pltpu.sync_copy(items_h.at[pl.ds(base, BPW)], i_v)

        tabs = [(mfu_h, r_mfu, 0), (mfi_h, r_mfi, 1),
                (mlpu_h, r_mlpu, 2), (mlpi_h, r_mlpi, 3)]
        iot = lax.iota(jnp.int32, V)

        sems = (sem0, sem1, sem2)

        def grp(g, _):
            ug = u_v[pl.ds(g * V, V)]
            ig = i_v[pl.ds(g * V, V)]
            cu = (ug // W) * W
            ci = (ig // W) * W
            lu = ug - cu
            li = ig - ci

            def fire(u):
                slot = u % 3
                cps = []
                for j in range(2):
                    l = u * 2 + j
                    for tab, r, t in tabs:
                        col = cu[l] if t in (0, 2) else ci[l]
                        col = pl.multiple_of(col, W)
                        cps.append(pltpu.async_copy(
                            tab.at[:, pl.ds(col, W)], r.at[slot * 2 + j],
                            sems[slot]))
                return cps

            def extract(u):
                slot = u % 3
                for j in range(2):
                    l = u * 2 + j
                    jvec = jnp.full((V,), slot * 2 + j, dtype=jnp.int32)
                    for tab, r, t in tabs:
                        lane = lu[l] if t in (0, 2) else li[l]
                        svec = jnp.full((V,), lane, dtype=jnp.int32)
                        for h in range(D // V):
                            vals = plsc.load_gather(
                                r, [jvec, h * V + iot, svec])
                            stage[l, pl.ds(t * D + h * V, V)] = vals

            pend0 = fire(0)
            pend1 = fire(1)
            for u in range(V // 2):
                nxt = fire(u + 2) if u < V // 2 - 2 else None
                for cp in pend0:
                    cp.wait()
                extract(u)
                pend0 = pend1
                pend1 = nxt
            pltpu.sync_copy(stage, out_h.at[pl.ds(base + g * V, V)])
            return 0

        lax.fori_loop(0, NG, grp, 0)

    return k(users, items, mf_uT, mf_iT, mlp_uT, mlp_iT)


def _tc_body(emb, w1a, w1b, b1r, w2, b2r, w3, b3r, woa, wob, bor, out):
    f32 = jnp.float32
    e = emb[:]
    gmf = e[:, :D] * e[:, D:2 * D]
    h = jnp.dot(e[:, 2 * D:3 * D], w1a[:], preferred_element_type=f32)
    h = h + jnp.dot(e[:, 3 * D:], w1b[:], preferred_element_type=f32)
    h = jnp.maximum(h + b1r[:], 0.0)
    h = jnp.maximum(jnp.dot(h, w2[:], preferred_element_type=f32) + b2r[:], 0.0)
    h = jnp.maximum(jnp.dot(h, w3[:], preferred_element_type=f32) + b3r[:], 0.0)
    logit = (jnp.dot(gmf, woa[:], preferred_element_type=f32)
             + jnp.dot(h, wob[:], preferred_element_type=f32) + bor[:])
    out[:] = jax.nn.sigmoid(logit)


def _tc_mlp(emb, W1, b1, W2, b2, W3, b3, Wo, bo):
    bs = 2048
    grid = (B // bs,)
    w1a, w1b = W1[:D], W1[D:]
    woa, wob = Wo[:D], Wo[D:]
    b1r = b1.reshape(1, -1)
    b2r = b2.reshape(1, -1)
    b3r = b3.reshape(1, -1)
    bor = bo.reshape(1, 1)

    def full(a):
        return pl.BlockSpec(a.shape, lambda i: (0,) * a.ndim)

    return pl.pallas_call(
        _tc_body,
        grid=grid,
        in_specs=[
            pl.BlockSpec((bs, F), lambda i: (i, 0)),
            full(w1a), full(w1b), full(b1r),
            full(W2), full(b2r),
            full(W3), full(b3r),
            full(woa), full(wob), full(bor),
        ],
        out_specs=pl.BlockSpec((bs, 1), lambda i: (i, 0)),
        out_shape=jax.ShapeDtypeStruct((B, 1), jnp.float32),
    )(emb, w1a, w1b, b1r, W2, b2r, W3, b3r, woa, wob, bor)


def kernel(users, items, mf_u, mf_i, mlp_u, mlp_i, W1, b1, W2, b2, W3, b3,
           Wo, bo):
    emb = _sc_gather(users, items, mf_u.T, mf_i.T, mlp_u.T, mlp_i.T)
    return _tc_mlp(emb, W1, b1, W2, b2, W3, b3, Wo, bo)
